# Initial kernel scaffold; baseline (speedup 1.0000x reference)
import jax
import jax.numpy as jnp
from jax.experimental import pallas as pl


def kernel(position_ids, table):
    raise NotImplementedError("write your pallas kernel here")


# speedup vs baseline: 2.1781x; 2.1781x over previous
import functools

import jax
import jax.numpy as jnp
from jax import lax
from jax.experimental import pallas as pl
from jax.experimental.pallas import tpu as pltpu
from jax.experimental.pallas import tpu_sc as plsc

DIM = 768
MAX_LEN = 8192
BATCH = 4
SEQ = 8192

NUM_CORES = 2
NUM_SUBCORES = 16
NUM_WORKERS = NUM_CORES * NUM_SUBCORES

B_TOTAL = BATCH * SEQ
B_PER_W = B_TOTAL // NUM_WORKERS
CHUNK = 64
N_CHUNKS = B_PER_W // CHUNK


def _sc_gather(table, idx_flat):
    mesh = plsc.VectorSubcoreMesh(core_axis_name="c", subcore_axis_name="s")

    @functools.partial(
        pl.kernel,
        mesh=mesh,
        out_type=jax.ShapeDtypeStruct((B_TOTAL, DIM), jnp.float32),
        scratch_types=[
            pltpu.VMEM((CHUNK,), jnp.int32),
            pltpu.VMEM((CHUNK, DIM), jnp.float32),
            pltpu.SemaphoreType.DMA,
        ],
    )
    def k(table_hbm, idx_hbm, out_hbm, idx_v, rows_v, sem):
        wid = lax.axis_index("s") * NUM_CORES + lax.axis_index("c")
        base = wid * B_PER_W

        @pl.loop(0, N_CHUNKS)
        def _(ci):
            off = base + ci * CHUNK
            pltpu.sync_copy(idx_hbm.at[pl.ds(off, CHUNK)], idx_v)
            pltpu.async_copy(table_hbm.at[idx_v], rows_v, sem).wait()
            pltpu.sync_copy(rows_v, out_hbm.at[pl.ds(off, CHUNK)])

    return k(table, idx_flat)


@jax.jit
def kernel(position_ids, table):
    idx_flat = position_ids.reshape(B_TOTAL).astype(jnp.int32)
    out = _sc_gather(table, idx_flat)
    return out.reshape(BATCH, SEQ, DIM)


# speedup vs baseline: 2.4639x; 1.1312x over previous
import functools

import jax
import jax.numpy as jnp
from jax import lax
from jax.experimental import pallas as pl
from jax.experimental.pallas import tpu as pltpu
from jax.experimental.pallas import tpu_sc as plsc

DIM = 768
MAX_LEN = 8192
BATCH = 4
SEQ = 8192

NUM_CORES = 2
NUM_SUBCORES = 16
NUM_WORKERS = NUM_CORES * NUM_SUBCORES

B_TOTAL = BATCH * SEQ
B_PER_W = B_TOTAL // NUM_WORKERS
CHUNK = 64
N_CHUNKS = B_PER_W // CHUNK


def _sc_gather(table, idx_flat):
    mesh = plsc.VectorSubcoreMesh(core_axis_name="c", subcore_axis_name="s")

    @functools.partial(
        pl.kernel,
        mesh=mesh,
        out_type=jax.ShapeDtypeStruct((B_TOTAL, DIM), jnp.float32),
        scratch_types=[
            pltpu.VMEM((B_PER_W,), jnp.int32),
            pltpu.VMEM((CHUNK, DIM), jnp.float32),
            pltpu.VMEM((CHUNK, DIM), jnp.float32),
            pltpu.SemaphoreType.DMA,
            pltpu.SemaphoreType.DMA,
            pltpu.SemaphoreType.DMA,
            pltpu.SemaphoreType.DMA,
        ],
    )
    def k(table_hbm, idx_hbm, out_hbm, idx_v, rows0, rows1, g0, g1, s0, s1):
        wid = lax.axis_index("s") * NUM_CORES + lax.axis_index("c")
        base = wid * B_PER_W
        pltpu.sync_copy(idx_hbm.at[pl.ds(base, B_PER_W)], idx_v)

        rows = [rows0, rows1]
        gsem = [g0, g1]
        ssem = [s0, s1]

        def gather_start(b, ci):
            return pltpu.async_copy(
                table_hbm.at[idx_v.at[pl.ds(ci * CHUNK, CHUNK)]], rows[b], gsem[b]
            )

        def store_start(b, ci):
            return pltpu.async_copy(
                rows[b], out_hbm.at[pl.ds(base + ci * CHUNK, CHUNK)], ssem[b]
            )

        g = [gather_start(0, 0), gather_start(1, 1)]
        s = [None, None]
        for ci in range(N_CHUNKS):
            b = ci % 2
            g[b].wait()
            s[b] = store_start(b, ci)
            if ci + 2 < N_CHUNKS:
                s[b].wait()
                g[b] = gather_start(b, ci + 2)
        s[0].wait()
        s[1].wait()

    return k(table, idx_flat)


@jax.jit
def kernel(position_ids, table):
    idx_flat = position_ids.reshape(B_TOTAL).astype(jnp.int32)
    out = _sc_gather(table, idx_flat)
    return out.reshape(BATCH, SEQ, DIM)


# speedup vs baseline: 2.5505x; 1.0351x over previous
import functools

import jax
import jax.numpy as jnp
from jax import lax
from jax.experimental import pallas as pl
from jax.experimental.pallas import tpu as pltpu
from jax.experimental.pallas import tpu_sc as plsc

DIM = 768
MAX_LEN = 8192
BATCH = 4
SEQ = 8192

NUM_CORES = 2
NUM_SUBCORES = 16
NUM_WORKERS = NUM_CORES * NUM_SUBCORES

B_TOTAL = BATCH * SEQ
B_PER_W = B_TOTAL // NUM_WORKERS
CHUNK = 32
N_CHUNKS = B_PER_W // CHUNK
NBUF = 4
N_GROUPS = N_CHUNKS // NBUF


def _sc_gather(table, idx_flat):
    mesh = plsc.VectorSubcoreMesh(core_axis_name="c", subcore_axis_name="s")

    @functools.partial(
        pl.kernel,
        mesh=mesh,
        out_type=jax.ShapeDtypeStruct((B_TOTAL, DIM), jnp.float32),
        scratch_types=[
            pltpu.VMEM((B_PER_W,), jnp.int32),
        ]
        + [pltpu.VMEM((CHUNK, DIM), jnp.float32)] * NBUF
        + [pltpu.SemaphoreType.DMA] * (2 * NBUF),
    )
    def k(table_hbm, idx_hbm, out_hbm, idx_v, *bufs):
        rows = list(bufs[:NBUF])
        gsem = list(bufs[NBUF : 2 * NBUF])
        ssem = list(bufs[2 * NBUF :])

        wid = lax.axis_index("s") * NUM_CORES + lax.axis_index("c")
        base = wid * B_PER_W
        pltpu.sync_copy(idx_hbm.at[pl.ds(base, B_PER_W)], idx_v)

        def gather_start(b, ci):
            return pltpu.async_copy(
                table_hbm.at[idx_v.at[pl.ds(ci * CHUNK, CHUNK)]], rows[b], gsem[b]
            )

        def gather_wait(b):
            pltpu.make_async_copy(
                table_hbm.at[idx_v.at[pl.ds(0, CHUNK)]], rows[b], gsem[b]
            ).wait()

        def store_start(b, ci):
            return pltpu.async_copy(
                rows[b], out_hbm.at[pl.ds(base + ci * CHUNK, CHUNK)], ssem[b]
            )

        for b in range(NBUF):
            gather_start(b, b)

        @pl.loop(0, N_GROUPS - 1)
        def _(j):
            cb = j * NBUF
            for b in range(NBUF):
                ci = cb + b
                gather_wait(b)
                store_start(b, ci).wait()
                gather_start(b, ci + NBUF)

        last = (N_GROUPS - 1) * NBUF
        s = []
        for b in range(NBUF):
            gather_wait(b)
            s.append(store_start(b, last + b))
        for h in s:
            h.wait()

    return k(table, idx_flat)


@jax.jit
def kernel(position_ids, table):
    idx_flat = position_ids.reshape(B_TOTAL).astype(jnp.int32)
    out = _sc_gather(table, idx_flat)
    return out.reshape(BATCH, SEQ, DIM)
